# baseline (speedup 1.0000x reference)
import functools

import jax
import jax.numpy as jnp
from jax import lax
from jax.experimental import pallas as pl
from jax.experimental.pallas import tpu as pltpu
from jax.experimental.pallas import tpu_sc as plsc


def _make_gather(V: int, D: int, B: int):
    info = plsc.get_sparse_core_info()
    NC, NS = info.num_cores, info.num_subcores
    NW = NC * NS
    assert B % (8 * NW) == 0 and D % info.num_lanes == 0
    b_per_w = B // NW
    mesh = plsc.VectorSubcoreMesh(core_axis_name="c", subcore_axis_name="s")

    @functools.partial(
        pl.kernel,
        mesh=mesh,
        out_type=jax.ShapeDtypeStruct((B, D), jnp.float32),
        scratch_types=[
            pltpu.VMEM((b_per_w,), jnp.int32),
            pltpu.VMEM((b_per_w, D), jnp.float32),
            pltpu.SemaphoreType.DMA,
        ],
    )
    def gather_k(idx_hbm, table_hbm, out_hbm, idx_v, rows_v, sem):
        wid = lax.axis_index("s") * NC + lax.axis_index("c")
        base = wid * b_per_w
        pltpu.sync_copy(idx_hbm.at[pl.ds(base, b_per_w)], idx_v)
        pltpu.async_copy(table_hbm.at[idx_v], rows_v, sem).wait()
        pltpu.sync_copy(rows_v, out_hbm.at[pl.ds(base, b_per_w)])

    return gather_k


def _head_body(x_ref, pos_ref, w_ref, out_ref):
    x = x_ref[...] + pos_ref[...]
    out_ref[...] = lax.dot_general(
        x, w_ref[...],
        dimension_numbers=(((1,), (1,)), ((), ())),
        preferred_element_type=jnp.float32,
    )


def _head(x, pos_emb, W_out, vt: int = 1024):
    S, D = x.shape
    V = W_out.shape[0]
    nv = pl.cdiv(V, vt)
    return pl.pallas_call(
        _head_body,
        grid=(nv,),
        in_specs=[
            pl.BlockSpec((S, D), lambda i: (0, 0)),
            pl.BlockSpec((S, D), lambda i: (0, 0)),
            pl.BlockSpec((vt, D), lambda i: (i, 0)),
        ],
        out_specs=pl.BlockSpec((S, vt), lambda i: (0, i)),
        out_shape=jax.ShapeDtypeStruct((S, V), jnp.float32),
    )(x, pos_emb, W_out)


def kernel(in_idx, tok_emb, pos_emb, W_out):
    b, s = in_idx.shape
    V, D = tok_emb.shape
    idx_flat = in_idx.reshape(b * s).astype(jnp.int32)
    x = _make_gather(V, D, b * s)(idx_flat, tok_emb)
    logits = _head(x, pos_emb[:s], W_out)
    return logits.reshape(b, s, W_out.shape[0])
